# baseline (speedup 1.0000x reference)
import jax
import jax.numpy as jnp
from jax.experimental import pallas as pl
from jax.experimental.pallas import tpu as pltpu

_N, _C, _H, _W = 4, 96, 224, 224
_CP = 128
_K = 3 * _CP
_WP = 256
_TROWS = 32
_TT = _TROWS * _WP
_NTD = _H // _TROWS
_NTF = _NTD + 2
_OFF = 2048
_FLAT = _OFF + _H * _WP + 2048
_TILE = 2048
_ROWS = _TILE // _WP
_NTC = _H // _ROWS
_PH = _NTF + _NTC
_CB = _TROWS // 4 * _W


def _body(x0_ref, x1_ref, x2_ref, x3_ref, w_ref, o_ref, s_ref):
    t = pl.program_id(1)

    @pl.when(t < _NTD)
    def _transform():
        ab = jnp.concatenate(
            [x0_ref[0], x1_ref[0], x2_ref[0], x3_ref[0]], axis=1
        ).astype(jnp.bfloat16)
        ga, gb, gc = [], [], []
        for h in range(_TROWS):
            lo, hi = h * _W, (h + 1) * _W
            ga.append(jnp.pad(
                jax.lax.slice(ab, (0, lo), (_C, hi - 1)),
                ((0, 0), (1, _WP - _W))))
            gb.append(jnp.pad(
                jax.lax.slice(ab, (0, lo), (_C, hi)),
                ((0, 0), (0, _WP - _W))))
            gc.append(jnp.pad(
                jax.lax.slice(ab, (0, lo + 1), (_C, hi)),
                ((0, 0), (0, _WP - _W + 1))))
        pre = jnp.concatenate([
            jnp.pad(jnp.concatenate(g, axis=1), ((0, _CP - _C), (0, 0)))
            for g in (ga, gb, gc)
        ], axis=0)
        v = jnp.transpose(pre, (1, 0))
        s_ref[pl.ds(_OFF + t * _TT, _TT), :] = v

    @pl.when(t == _NTD)
    def _bottom_guard():
        s_ref[pl.ds(_OFF + _H * _WP, 2048), :] = jnp.zeros(
            (2048, _K), jnp.bfloat16)

    @pl.when(t == _NTD + 1)
    def _top_guard():
        s_ref[pl.ds(0, _OFF), :] = jnp.zeros((_OFF, _K), jnp.bfloat16)

    @pl.when(t >= _NTF)
    def _conv():
        base = jnp.maximum(t - _NTF, 0) * _TILE
        out = jnp.zeros((_TILE, _C), jnp.float32)
        for kh in range(3):
            slab = s_ref[pl.ds(base + _OFF - _WP + kh * _WP, _TILE), :]
            out = out + jax.lax.dot_general(
                slab, w_ref[kh],
                (((1,), (0,)), ((), ())),
                preferred_element_type=jnp.float32,
            )
        outT = jnp.transpose(out, (1, 0)).reshape(_C, _ROWS, _WP)
        o_ref[0] = jax.lax.slice(outT, (0, 0, 0), (_C, _ROWS, _W))


def kernel(input, W):
    x2 = input.reshape(_N, _C, _H * _W)
    wt = jnp.transpose(W, (2, 3, 1, 0))
    wt = jnp.pad(wt, ((0, 0), (0, 0), (0, _CP - _C), (0, 0)))
    wcat = wt.reshape(3, _K, _C).astype(jnp.bfloat16)

    def _in_spec(k):
        return pl.BlockSpec(
            (1, _C, _CB),
            lambda n, t, k=k: (n, 0, jnp.clip(4 * t + k, 0, 4 * _NTD - 1)),
        )

    y = pl.pallas_call(
        _body,
        grid=(_N, _PH),
        in_specs=[_in_spec(0), _in_spec(1), _in_spec(2), _in_spec(3),
                  pl.BlockSpec((3, _K, _C), lambda n, t: (0, 0, 0))],
        out_specs=pl.BlockSpec(
            (1, _C, _ROWS, _W),
            lambda n, t: (n, 0, jnp.clip(t - _NTF, 0, _NTC - 1), 0)),
        out_shape=jax.ShapeDtypeStruct((_N, _C, _H, _W), jnp.float32),
        scratch_shapes=[pltpu.VMEM((_FLAT, _K), jnp.bfloat16)],
    )(x2, x2, x2, x2, wcat)
    return y


# speedup vs baseline: 1.1495x; 1.1495x over previous
import jax
import jax.numpy as jnp
from jax.experimental import pallas as pl
from jax.experimental.pallas import tpu as pltpu

_N, _C, _H, _W = 4, 96, 224, 224
_CP = 128
_WP = 256
_TROWS = 32
_TT = _TROWS * _WP
_NTF = _H // _TROWS + 2
_FLAT = _NTF * _TT
_OFF = _TT
_TILE = 2048
_CHUNK = 512
_ROWS = _TILE // _WP
_NTC = _H // _ROWS
_PH = _NTF + _NTC


def _body(x_ref, w_ref, o_ref, s_ref):
    t = pl.program_id(1)

    @pl.when(t < _NTF)
    def _transform():
        a = x_ref[0]
        at = jnp.transpose(a, (1, 0))
        pieces = [
            jnp.pad(
                jax.lax.slice(at, (h * _W, 0), ((h + 1) * _W, _C)),
                ((0, _WP - _W), (0, _CP - _C)),
            )
            for h in range(_TROWS)
        ]
        v = jnp.concatenate(pieces, axis=0)
        valid = jnp.logical_and(t >= 1, t <= _NTF - 2)
        v = jnp.where(valid, v, jnp.zeros_like(v))
        s_ref[pl.ds(jnp.minimum(t, _NTF - 1) * _TT, _TT), :] = v.astype(
            jnp.bfloat16)

    @pl.when(t >= _NTF)
    def _conv():
        base = jnp.maximum(t - _NTF, 0) * _TILE
        slabs = [
            s_ref[pl.ds(base + _OFF - _WP - 16 + kh * _WP, _TILE + 24), :]
            for kh in range(3)
        ]
        cat = jnp.concatenate(slabs, axis=1)
        out = jnp.zeros((_TILE, _C), jnp.float32)
        for kw in range(3):
            p = jax.lax.dot_general(
                cat, w_ref[kw],
                (((1,), (0,)), ((), ())),
                preferred_element_type=jnp.float32,
            )
            out = out + jax.lax.slice(p, (15 + kw, 0), (15 + kw + _TILE, _C))
        outT = jnp.transpose(out, (1, 0)).reshape(_C, _ROWS, _WP)
        o_ref[0] = jax.lax.slice(outT, (0, 0, 0), (_C, _ROWS, _W))


def kernel(input, W):
    x2 = input.reshape(_N, _C, _H * _W)
    wt = jnp.transpose(W, (2, 3, 1, 0))
    wt = jnp.pad(wt, ((0, 0), (0, 0), (0, _CP - _C), (0, 0)))
    wcat = jnp.transpose(wt, (1, 0, 2, 3)).reshape(3, 3 * _CP, _C)
    wcat = wcat.astype(jnp.bfloat16)
    y = pl.pallas_call(
        _body,
        grid=(_N, _PH),
        in_specs=[
            pl.BlockSpec(
                (1, _C, _TROWS * _W),
                lambda n, t: (n, 0, jnp.clip(t - 1, 0, _H // _TROWS - 1)),
            ),
            pl.BlockSpec((3, 3 * _CP, _C), lambda n, t: (0, 0, 0)),
        ],
        out_specs=pl.BlockSpec(
            (1, _C, _ROWS, _W),
            lambda n, t: (n, 0, jnp.clip(t - _NTF, 0, _NTC - 1), 0)),
        out_shape=jax.ShapeDtypeStruct((_N, _C, _H, _W), jnp.float32),
        scratch_shapes=[pltpu.VMEM((_FLAT, _CP), jnp.bfloat16)],
    )(x2, wcat)
    return y


# speedup vs baseline: 1.1502x; 1.0006x over previous
import jax
import jax.numpy as jnp
from jax.experimental import pallas as pl
from jax.experimental.pallas import tpu as pltpu

_N, _C, _H, _W = 4, 96, 224, 224
_CP = 128
_WP = 256
_TROWS = 32
_TT = _TROWS * _WP
_NTF = _H // _TROWS + 2
_FLAT = _NTF * _TT
_OFF = _TT
_TILE = 2048
_ROWS = _TILE // _WP
_NTC = _H // _ROWS
_PH = _NTF + _NTC


def _body(x_ref, w_ref, o_ref, s_ref):
    t = pl.program_id(1)

    @pl.when(t < _NTF)
    def _transform():
        a = x_ref[0]
        at = jnp.transpose(a, (1, 0))
        pieces = [
            jnp.pad(
                jax.lax.slice(at, (h * _W, 0), ((h + 1) * _W, _C)),
                ((0, _WP - _W), (0, _CP - _C)),
            )
            for h in range(_TROWS)
        ]
        v = jnp.concatenate(pieces, axis=0)
        valid = jnp.logical_and(t >= 1, t <= _NTF - 2)
        v = jnp.where(valid, v, jnp.zeros_like(v))
        s_ref[pl.ds(jnp.minimum(t, _NTF - 1) * _TT, _TT), :] = v.astype(
            jnp.bfloat16)

    @pl.when(t >= _NTF)
    def _conv():
        base = jnp.maximum(t - _NTF, 0) * _TILE
        slabs = [
            s_ref[pl.ds(base + _OFF - _WP - 16 + kh * _WP, _TILE + 24), :]
            for kh in range(3)
        ]
        cat = jnp.concatenate(slabs, axis=1)
        out = jnp.zeros((_TILE, _C), jnp.float32)
        for kw in range(3):
            p = jax.lax.dot_general(
                cat, w_ref[kw],
                (((1,), (0,)), ((), ())),
                preferred_element_type=jnp.float32,
            )
            out = out + jax.lax.slice(p, (15 + kw, 0), (15 + kw + _TILE, _C))
        outT = jnp.transpose(out, (1, 0)).reshape(_C, _ROWS, _WP)
        o_ref[0] = jax.lax.slice(outT, (0, 0, 0), (_C, _ROWS, _W))


def kernel(input, W):
    x2 = input.reshape(_N, _C, _H * _W)
    wt = jnp.transpose(W, (2, 3, 1, 0))
    wt = jnp.pad(wt, ((0, 0), (0, 0), (0, _CP - _C), (0, 0)))
    wcat = jnp.transpose(wt, (1, 0, 2, 3)).reshape(3, 3 * _CP, _C)
    wcat = wcat.astype(jnp.bfloat16)
    y = pl.pallas_call(
        _body,
        grid=(_N, _PH),
        in_specs=[
            pl.BlockSpec(
                (1, _C, _TROWS * _W),
                lambda n, t: (n, 0, jnp.clip(t - 1, 0, _H // _TROWS - 1)),
            ),
            pl.BlockSpec((3, 3 * _CP, _C), lambda n, t: (0, 0, 0)),
        ],
        out_specs=pl.BlockSpec(
            (1, _C, _ROWS, _W),
            lambda n, t: (n, 0, jnp.clip(t - _NTF, 0, _NTC - 1), 0)),
        out_shape=jax.ShapeDtypeStruct((_N, _C, _H, _W), jnp.float32),
        scratch_shapes=[pltpu.VMEM((_FLAT, _CP), jnp.bfloat16)],
    )(x2, wcat)
    return y
